# baseline (speedup 1.0000x reference)
import functools

import jax
import jax.numpy as jnp
from jax import lax
from jax.experimental import pallas as pl
from jax.experimental.pallas import tpu as pltpu
from jax.experimental.pallas import tpu_sc as plsc

NCORES = 2
NSUB = 16
NW = NCORES * NSUB
LANES = 16
CHUNK = 16
NBINS = 65
NFREQ = 64


def _make_hist_fn(nrows, rowlen, interpret=False):
    assert nrows % (NW * CHUNK) == 0
    rows_per_w = nrows // NW
    nchunks = rows_per_w // CHUNK
    assert nchunks % 2 == 0
    assert rowlen % LANES == 0 and (rowlen & (rowlen - 1)) == 0

    ctiles = rowlen // 128

    mesh = plsc.VectorSubcoreMesh(
        core_axis_name="c", subcore_axis_name="s",
        num_cores=NCORES, num_subcores=NSUB)

    @functools.partial(
        pl.kernel,
        out_type=jax.ShapeDtypeStruct((nrows * NFREQ,), jnp.float32),
        mesh=mesh,
        scratch_types=[
            pltpu.VMEM((2, ctiles, 8, 128), jnp.int32),
            pltpu.VMEM((2, ctiles, 8, 128), jnp.int32),
            pltpu.VMEM((CHUNK * NBINS,), jnp.float32),
            pltpu.VMEM((CHUNK * NFREQ,), jnp.float32),
            pltpu.SemaphoreType.DMA,
            pltpu.SemaphoreType.DMA,
        ],
        compiler_params=pltpu.CompilerParams(needs_layout_passes=False),
        interpret=interpret,
    )
    def hist_kernel(rna_hbm, freq_hbm, data0, data1, hist, fbuf, sem0, sem1):
        c = lax.axis_index("c")
        s = lax.axis_index("s")
        wid = s * NCORES + c
        row0 = wid * rows_per_w

        lane = lax.broadcasted_iota(jnp.int32, (LANES,), 0)
        lane_tr = lane >> 3
        lane_sr = lane & 7
        ones = jnp.full((LANES,), 1.0, jnp.float32)
        zeros = jnp.zeros((LANES,), jnp.float32)

        def dma_in(chunk_idx, data_ref, sem):
            tr0 = (row0 + chunk_idx * CHUNK) // 8
            return pltpu.make_async_copy(
                rna_hbm.at[pl.ds(tr0, 2)], data_ref, sem)

        def process(chunk_idx, data_ref):
            for k in range(NBINS):
                hist[pl.ds(k * LANES, LANES)] = zeros

            def ct_body(ct, carry):
                i1 = jnp.zeros((LANES,), jnp.int32) + ct

                @plsc.parallel_loop(0, 128, 1, unroll=16)
                def _(w):
                    i3 = (lane + w) & 127
                    v = plsc.load_gather(data_ref, [lane_tr, i1, lane_sr, i3])
                    plsc.addupdate_scatter(hist, [(v << 4) | lane], ones)

                return carry
            lax.fori_loop(0, ctiles, ct_body, 0)

            for r in range(CHUNK):
                cs = []
                for k in range(4):
                    idx = (lane << 4) + (r + 16 * (1 + 16 * k))
                    cs.append(plsc.load_gather(hist, [idx]))
                total = jnp.sum(cs[0] + cs[1] + cs[2] + cs[3])
                for k in range(4):
                    fbuf[pl.ds(r * NFREQ + k * 16, 16)] = cs[k] / total

            out_start = (row0 + chunk_idx * CHUNK) * NFREQ
            out_start = pl.multiple_of(out_start, 8)
            pltpu.sync_copy(fbuf, freq_hbm.at[pl.ds(out_start, CHUNK * NFREQ)])

        dma_in(0, data0, sem0).start()
        npairs = nchunks // 2

        def pair_body(i, carry):
            cb = i * 2
            dma_in(cb, data0, sem0).wait()
            dma_in(cb + 1, data1, sem1).start()
            process(cb, data0)
            dma_in(cb + 1, data1, sem1).wait()

            @pl.when(i + 1 < npairs)
            def _():
                dma_in(cb + 2, data0, sem0).start()

            process(cb + 1, data1)
            return carry

        lax.fori_loop(0, npairs, pair_body, 0)

    return hist_kernel


def _make_mlp_fn(nrows, hid, block_rows, interpret=False):
    assert nrows % block_rows == 0
    grid = nrows // block_rows

    def mlp_kernel(freq_ref, tid_ref, ttab_ref, w1a_ref, w1b_ref,
                   b1_ref, w2_ref, b2_ref, y_ref):
        tt = ttab_ref[...]
        nrm = jnp.sqrt(jnp.sum(tt * tt, axis=1, keepdims=True))
        scale = jnp.where(nrm > 1.0, 1.0 / jnp.maximum(nrm, 1e-7), 1.0)
        tn = tt * scale
        m = jnp.dot(tn, w1a_ref[...], preferred_element_type=jnp.float32)

        tid = tid_ref[...]
        onehot = (tid == lax.broadcasted_iota(jnp.int32, (1, NFREQ), 1)
                  ).astype(jnp.float32)
        h = (jnp.dot(onehot, m, preferred_element_type=jnp.float32)
             + jnp.dot(freq_ref[...], w1b_ref[...],
                       preferred_element_type=jnp.float32)
             + b1_ref[...])
        h = jnp.maximum(h, 0.0)
        y_ref[...] = jnp.sum(h * w2_ref[...], axis=1, keepdims=True) + b2_ref[0, 0]

    return pl.pallas_call(
        mlp_kernel,
        grid=(grid,),
        in_specs=[
            pl.BlockSpec((block_rows, NFREQ), lambda i: (i, 0)),
            pl.BlockSpec((block_rows, 1), lambda i: (i, 0)),
            pl.BlockSpec((NFREQ, NFREQ), lambda i: (0, 0)),
            pl.BlockSpec((NFREQ, hid), lambda i: (0, 0)),
            pl.BlockSpec((NFREQ, hid), lambda i: (0, 0)),
            pl.BlockSpec((1, hid), lambda i: (0, 0)),
            pl.BlockSpec((1, hid), lambda i: (0, 0)),
            pl.BlockSpec((1, 1), lambda i: (0, 0)),
        ],
        out_specs=pl.BlockSpec((block_rows, 1), lambda i: (i, 0)),
        out_shape=jax.ShapeDtypeStruct((nrows, 1), jnp.float32),
        interpret=interpret,
    )


def kernel(rna_data, tissue_id, tissue_table, W1, b1, W2, b2):
    nrows, rowlen = rna_data.shape
    hid = W1.shape[1]

    hist_fn = _make_hist_fn(nrows, rowlen)
    rna_tiles = rna_data.reshape(nrows // 8, rowlen // 128, 8, 128)
    freq = hist_fn(rna_tiles).reshape(nrows, NFREQ)

    mlp_fn = _make_mlp_fn(nrows, hid, block_rows=1024)
    y = mlp_fn(
        freq,
        tissue_id.reshape(nrows, 1),
        tissue_table,
        W1[:NFREQ],
        W1[NFREQ:],
        b1.reshape(1, hid),
        W2.reshape(1, hid),
        b2.reshape(1, 1),
    )
    return y


# speedup vs baseline: 1.5963x; 1.5963x over previous
import functools

import jax
import jax.numpy as jnp
from jax import lax
from jax.experimental import pallas as pl
from jax.experimental.pallas import tpu as pltpu
from jax.experimental.pallas import tpu_sc as plsc

NCORES = 2
NSUB = 16
NW = NCORES * NSUB
LANES = 16
CHUNK = 16
NBINS = 65
NFREQ = 64


def _make_hist_fn(nrows, rowlen, interpret=False):
    assert nrows % (NW * CHUNK) == 0
    rows_per_w = nrows // NW
    nchunks = rows_per_w // CHUNK
    assert nchunks % 2 == 0
    assert rowlen % LANES == 0 and (rowlen & (rowlen - 1)) == 0

    ctiles = rowlen // 128

    mesh = plsc.VectorSubcoreMesh(
        core_axis_name="c", subcore_axis_name="s",
        num_cores=NCORES, num_subcores=NSUB)

    @functools.partial(
        pl.kernel,
        out_type=jax.ShapeDtypeStruct((nrows * NFREQ,), jnp.float32),
        mesh=mesh,
        scratch_types=[
            pltpu.VMEM((CHUNK, rowlen), jnp.int32),
            pltpu.VMEM((CHUNK, rowlen), jnp.int32),
            pltpu.VMEM((CHUNK * NBINS,), jnp.float32),
            pltpu.VMEM((CHUNK * NFREQ,), jnp.float32),
            pltpu.SemaphoreType.DMA,
            pltpu.SemaphoreType.DMA,
        ],
        compiler_params=pltpu.CompilerParams(
            needs_layout_passes=False, use_tc_tiling_on_sc=True),
        interpret=interpret,
    )
    def hist_kernel(rna_hbm, freq_hbm, data0, data1, hist, fbuf, sem0, sem1):
        c = lax.axis_index("c")
        s = lax.axis_index("s")
        wid = s * NCORES + c
        row0 = wid * rows_per_w

        lane = lax.broadcasted_iota(jnp.int32, (LANES,), 0)
        ones = jnp.full((LANES,), 1.0, jnp.float32)
        zeros = jnp.zeros((LANES,), jnp.float32)

        def dma_in(chunk_idx, data_ref, sem):
            r0 = row0 + chunk_idx * CHUNK
            return pltpu.make_async_copy(
                rna_hbm.at[pl.ds(r0, CHUNK)], data_ref, sem)

        def process(chunk_idx, data_ref):
            for k in range(NBINS):
                hist[pl.ds(k * LANES, LANES)] = zeros

            @plsc.parallel_loop(0, rowlen, 1, unroll=16)
            def _(j):
                col = (lane + j) & (rowlen - 1)
                v = plsc.load_gather(data_ref, [lane, col])
                plsc.addupdate_scatter(hist, [(v << 4) | lane], ones)

            for r in range(CHUNK):
                cs = []
                for k in range(4):
                    idx = (lane << 4) + (r + 16 * (1 + 16 * k))
                    cs.append(plsc.load_gather(hist, [idx]))
                total = jnp.sum(cs[0] + cs[1] + cs[2] + cs[3])
                for k in range(4):
                    fbuf[pl.ds(r * NFREQ + k * 16, 16)] = cs[k] / total

            out_start = (row0 + chunk_idx * CHUNK) * NFREQ
            out_start = pl.multiple_of(out_start, 8)
            pltpu.sync_copy(fbuf, freq_hbm.at[pl.ds(out_start, CHUNK * NFREQ)])

        dma_in(0, data0, sem0).start()
        npairs = nchunks // 2

        def pair_body(i, carry):
            cb = i * 2
            dma_in(cb, data0, sem0).wait()
            dma_in(cb + 1, data1, sem1).start()
            process(cb, data0)
            dma_in(cb + 1, data1, sem1).wait()

            @pl.when(i + 1 < npairs)
            def _():
                dma_in(cb + 2, data0, sem0).start()

            process(cb + 1, data1)
            return carry

        lax.fori_loop(0, npairs, pair_body, 0)

    return hist_kernel


def _make_mlp_fn(nrows, hid, block_rows, interpret=False):
    assert nrows % block_rows == 0
    grid = nrows // block_rows

    def mlp_kernel(freq_ref, tid_ref, ttab_ref, w1a_ref, w1b_ref,
                   b1_ref, w2_ref, b2_ref, y_ref):
        tt = ttab_ref[...]
        nrm = jnp.sqrt(jnp.sum(tt * tt, axis=1, keepdims=True))
        scale = jnp.where(nrm > 1.0, 1.0 / jnp.maximum(nrm, 1e-7), 1.0)
        tn = tt * scale
        m = jnp.dot(tn, w1a_ref[...], preferred_element_type=jnp.float32)

        tid = tid_ref[...]
        onehot = (tid == lax.broadcasted_iota(jnp.int32, (1, NFREQ), 1)
                  ).astype(jnp.float32)
        h = (jnp.dot(onehot, m, preferred_element_type=jnp.float32)
             + jnp.dot(freq_ref[...], w1b_ref[...],
                       preferred_element_type=jnp.float32)
             + b1_ref[...])
        h = jnp.maximum(h, 0.0)
        y_ref[...] = jnp.sum(h * w2_ref[...], axis=1, keepdims=True) + b2_ref[0, 0]

    return pl.pallas_call(
        mlp_kernel,
        grid=(grid,),
        in_specs=[
            pl.BlockSpec((block_rows, NFREQ), lambda i: (i, 0)),
            pl.BlockSpec((block_rows, 1), lambda i: (i, 0)),
            pl.BlockSpec((NFREQ, NFREQ), lambda i: (0, 0)),
            pl.BlockSpec((NFREQ, hid), lambda i: (0, 0)),
            pl.BlockSpec((NFREQ, hid), lambda i: (0, 0)),
            pl.BlockSpec((1, hid), lambda i: (0, 0)),
            pl.BlockSpec((1, hid), lambda i: (0, 0)),
            pl.BlockSpec((1, 1), lambda i: (0, 0)),
        ],
        out_specs=pl.BlockSpec((block_rows, 1), lambda i: (i, 0)),
        out_shape=jax.ShapeDtypeStruct((nrows, 1), jnp.float32),
        interpret=interpret,
    )


def kernel(rna_data, tissue_id, tissue_table, W1, b1, W2, b2):
    nrows, rowlen = rna_data.shape
    hid = W1.shape[1]

    hist_fn = _make_hist_fn(nrows, rowlen)
    freq = hist_fn(rna_data).reshape(nrows, NFREQ)

    mlp_fn = _make_mlp_fn(nrows, hid, block_rows=1024)
    y = mlp_fn(
        freq,
        tissue_id.reshape(nrows, 1),
        tissue_table,
        W1[:NFREQ],
        W1[NFREQ:],
        b1.reshape(1, hid),
        W2.reshape(1, hid),
        b2.reshape(1, 1),
    )
    return y
